# baseline (speedup 1.0000x reference)
import jax
import jax.numpy as jnp
from jax import lax
from jax.experimental import pallas as pl
from jax.experimental.pallas import tpu as pltpu
from jax.experimental.pallas import tpu_sc as plsc

_NUM_EMBEDDINGS = 8192
_D = 64
_BS = 131072
_NC = 2
_NS = 16
_NW = _NC * _NS
_NH = 2
_HS = _BS // _NH
_BW = _HS // _NW
_C = 512
_NCHUNK = _BW // _C
_NBUF = 3

_TB = 32768
_NTB = _HS // _TB


def _make_gather_body(h):
  def _gather_body(cat_hbm, cb_hbm, q_hbm, idx_v, rows_v, sem_g, sem_o):
    wid = lax.axis_index("s") * _NC + lax.axis_index("c")
    base = h * _HS + wid * _BW

    pltpu.sync_copy(cat_hbm.at[pl.ds(base, _BW)], idx_v)

    def q_dst(g):
      t0 = wid * _BW + g * _C
      blk = t0 // _TB
      j = t0 % _TB
      half = j // (_TB // 2)
      row0 = blk * (_TB // 2) + j % (_TB // 2)
      return q_hbm.at[pl.ds(row0, _C), pl.ds(half * _D, _D)]

    def fire(g):
      b = g % _NBUF
      pltpu.async_copy(cb_hbm.at[idx_v.at[pl.ds(g * _C, _C)]],
                       rows_v.at[b], sem_g.at[b])

    def wait_g(g):
      b = g % _NBUF
      pltpu.make_async_copy(cb_hbm.at[idx_v.at[pl.ds(g * _C, _C)]],
                            rows_v.at[b], sem_g.at[b]).wait()

    def fire_out(g):
      b = g % _NBUF
      pltpu.async_copy(rows_v.at[b], q_dst(g), sem_o.at[b])

    def wait_out(g):
      b = g % _NBUF
      pltpu.make_async_copy(rows_v.at[b], q_dst(g), sem_o.at[b]).wait()

    fire(0)
    fire(1)
    for g in range(_NCHUNK):
      wait_g(g)
      fire_out(g)
      if g + 2 < _NCHUNK:
        if g >= 1:
          wait_out(g - 1)
        fire(g + 2)
    for g in range(max(0, _NCHUNK - 3), _NCHUNK):
      wait_out(g)

  return _gather_body


def _loss_body(q_ref, x_ref, o_ref, loss_ref):
  i = pl.program_id(0)

  @pl.when(i == 0)
  def _():
    loss_ref[0] = 0.0

  qt = lax.transpose(q_ref[...], (1, 0))
  a = qt[0:_D, :]
  b = qt[_D:2 * _D, :]
  o_ref[:, 0:_TB // 2] = a
  o_ref[:, _TB // 2:_TB] = b
  x = x_ref[...]
  d1 = a - x[:, 0:_TB // 2]
  d2 = b - x[:, _TB // 2:_TB]
  loss_ref[0] += jnp.sum(d1 * d1) + jnp.sum(d2 * d2)


@jax.jit
def kernel(inputs, categories, codebook):
  mesh = plsc.VectorSubcoreMesh(
      core_axis_name="c", subcore_axis_name="s",
      num_cores=_NC, num_subcores=_NS)

  def sc_gather(h):
    return pl.kernel(
        _make_gather_body(h),
        out_type=jax.ShapeDtypeStruct((_HS // 2, 128), jnp.float32),
        mesh=mesh,
        compiler_params=pltpu.CompilerParams(use_tc_tiling_on_sc=False),
        scratch_types=[
            pltpu.VMEM((_BW,), jnp.int32),
            pltpu.VMEM((_NBUF, _C, _D), jnp.float32),
            pltpu.SemaphoreType.DMA((_NBUF,)),
            pltpu.SemaphoreType.DMA((_NBUF,)),
        ],
    )(categories, codebook)

  x_t = inputs.T

  def tc_loss(h, q_h, out_init):
    in_specs = [
        pl.BlockSpec((_TB // 2, 128), lambda i: (i, 0)),
        pl.BlockSpec((_D, _TB), lambda i, h=h: (0, i + h * _NTB)),
    ]
    args = [q_h, x_t]
    aliases = {}
    if out_init is not None:
      in_specs.append(pl.BlockSpec(memory_space=pl.ANY))
      args.append(out_init)
      aliases = {2: 0}

    def body(q_ref, x_ref, *rest):
      o_ref, loss_ref = rest[-2], rest[-1]
      _loss_body(q_ref, x_ref, o_ref, loss_ref)

    return pl.pallas_call(
        body,
        grid=(_NTB,),
        in_specs=in_specs,
        out_specs=[
            pl.BlockSpec((_D, _TB), lambda i, h=h: (0, i + h * _NTB)),
            pl.BlockSpec(memory_space=pltpu.SMEM),
        ],
        out_shape=[
            jax.ShapeDtypeStruct((_D, _BS), jnp.float32),
            jax.ShapeDtypeStruct((1,), jnp.float32),
        ],
        input_output_aliases=aliases,
        compiler_params=pltpu.CompilerParams(
            dimension_semantics=("arbitrary",)),
    )(*args)

  q0 = sc_gather(0)
  q1 = sc_gather(1)
  out_t, tot0 = tc_loss(0, q0, None)
  out_t, tot1 = tc_loss(1, q1, out_t)
  loss = (tot0[0] + tot1[0]) * (0.5 / (_BS * _D))
  return out_t.T, loss


# speedup vs baseline: 1.0163x; 1.0163x over previous
import jax
import jax.numpy as jnp
from jax import lax
from jax.experimental import pallas as pl
from jax.experimental.pallas import tpu as pltpu
from jax.experimental.pallas import tpu_sc as plsc

_NUM_EMBEDDINGS = 8192
_D = 64
_BS = 131072
_NC = 2
_NS = 16
_NW = _NC * _NS
_BW = _BS // _NW
_C = 512
_NCHUNK = _BW // _C
_NBUF = 3

_TB = 32768
_NTB = _BS // _TB


def _gather_body(cat_hbm, cb_hbm, q_hbm, idx_v, rows_v, sem_g, sem_o):
  wid = lax.axis_index("s") * _NC + lax.axis_index("c")
  base = wid * _BW

  pltpu.sync_copy(cat_hbm.at[pl.ds(base, _BW)], idx_v)

  def q_dst(g):
    t0 = base + g * _C
    blk = t0 // _TB
    j = t0 % _TB
    half = j // (_TB // 2)
    row0 = blk * (_TB // 2) + j % (_TB // 2)
    return q_hbm.at[pl.ds(row0, _C), pl.ds(half * _D, _D)]

  def fire(g):
    b = g % _NBUF
    pltpu.async_copy(cb_hbm.at[idx_v.at[pl.ds(g * _C, _C)]],
                     rows_v.at[b], sem_g.at[b])

  def wait_g(g):
    b = g % _NBUF
    pltpu.make_async_copy(cb_hbm.at[idx_v.at[pl.ds(g * _C, _C)]],
                          rows_v.at[b], sem_g.at[b]).wait()

  def fire_out(g):
    b = g % _NBUF
    pltpu.async_copy(rows_v.at[b], q_dst(g), sem_o.at[b])

  def wait_out(g):
    b = g % _NBUF
    pltpu.make_async_copy(rows_v.at[b], q_dst(g), sem_o.at[b]).wait()

  fire(0)
  fire(1)
  for g in range(_NCHUNK):
    wait_g(g)
    fire_out(g)
    if g + 2 < _NCHUNK:
      if g >= 1:
        wait_out(g - 1)
      fire(g + 2)
  for g in range(_NCHUNK - 3, _NCHUNK):
    wait_out(g)


def _loss_body(q_ref, x_ref, o_ref, loss_ref):
  i = pl.program_id(0)

  @pl.when(i == 0)
  def _():
    loss_ref[0] = 0.0

  qt = lax.transpose(q_ref[...], (1, 0))
  a = qt[0:_D, :]
  b = qt[_D:2 * _D, :]
  o_ref[:, 0:_TB // 2] = a
  o_ref[:, _TB // 2:_TB] = b
  x = x_ref[...]
  d1 = a - x[:, 0:_TB // 2]
  d2 = b - x[:, _TB // 2:_TB]
  loss_ref[0] += jnp.sum(d1 * d1) + jnp.sum(d2 * d2)

  @pl.when(i == _NTB - 1)
  def _():
    loss_ref[0] = loss_ref[0] * (0.5 / (_BS * _D))


@jax.jit
def kernel(inputs, categories, codebook):
  mesh = plsc.VectorSubcoreMesh(
      core_axis_name="c", subcore_axis_name="s",
      num_cores=_NC, num_subcores=_NS)
  q = pl.kernel(
      _gather_body,
      out_type=jax.ShapeDtypeStruct((_BS // 2, 128), jnp.float32),
      mesh=mesh,
      compiler_params=pltpu.CompilerParams(use_tc_tiling_on_sc=False),
      scratch_types=[
          pltpu.VMEM((_BW,), jnp.int32),
          pltpu.VMEM((_NBUF, _C, _D), jnp.float32),
          pltpu.SemaphoreType.DMA((_NBUF,)),
          pltpu.SemaphoreType.DMA((_NBUF,)),
      ],
  )(categories, codebook)

  out_t, tot = pl.pallas_call(
      _loss_body,
      grid=(_NTB,),
      in_specs=[
          pl.BlockSpec((_TB // 2, 128), lambda i: (i, 0)),
          pl.BlockSpec((_D, _TB), lambda i: (0, i)),
      ],
      out_specs=[
          pl.BlockSpec((_D, _TB), lambda i: (0, i)),
          pl.BlockSpec(memory_space=pltpu.SMEM),
      ],
      out_shape=[
          jax.ShapeDtypeStruct((_D, _BS), jnp.float32),
          jax.ShapeDtypeStruct((1,), jnp.float32),
      ],
      compiler_params=pltpu.CompilerParams(
          dimension_semantics=("arbitrary",)),
  )(q, inputs.T)

  return out_t.T, tot[0]
